# baseline (speedup 1.0000x reference)
import functools

import jax
import jax.numpy as jnp
from jax import lax
from jax.experimental import pallas as pl
from jax.experimental.pallas import tpu as pltpu
from jax.experimental.pallas import tpu_sc as plsc

N = 10000
D_FEAT = 128
DIM = 64
Q = 3
N_LAYERS = 4
E_PER = 320000

CHUNK = 128
CHUNKS_PER_LAYER = E_PER // CHUNK
NC, NS = 2, 16
CPT = 160
CHUNKS_PAD = CPT * NS
ACC_ROWS = 10240
ZROWS = ACC_ROWS // NS


def _proj_body(f_ref, g_ref, o_ref):
    f = f_ref[...]
    ss = jnp.sum(f * f, axis=1, keepdims=True)
    fn = f / (jnp.sqrt(ss) + 1e-12)
    o_ref[...] = jnp.dot(fn, g_ref[...], preferred_element_type=jnp.float32)


def _project(feature, G):
    blk = 1000
    grid = N // blk
    return pl.pallas_call(
        _proj_body,
        grid=(grid,),
        in_specs=[
            pl.BlockSpec((blk, D_FEAT), lambda i: (i, 0)),
            pl.BlockSpec((D_FEAT, DIM), lambda i: (0, 0)),
        ],
        out_specs=pl.BlockSpec((blk, DIM), lambda i: (i, 0)),
        out_shape=jax.ShapeDtypeStruct((N, DIM), jnp.float32),
    )(feature, G)


NBUF = 4
IG = 32


def _hop_body(u_hbm, src_hbm, dst_hbm, zeros_hbm, p_hbm,
              sidx, didx, rows, acc0, acc1, gsem, ssem):
    c = lax.axis_index("c")
    t = lax.axis_index("s")

    pltpu.sync_copy(zeros_hbm, acc0.at[pl.ds(t * ZROWS, ZROWS)])
    pltpu.sync_copy(zeros_hbm, acc1.at[pl.ds(t * ZROWS, ZROWS)])
    plsc.subcore_barrier()

    for ll in range(2):
        acc = acc0 if ll == 0 else acc1
        layer = c * 2 + ll
        for ig in range(CPT // IG):
            base = t * CPT + ig * IG
            pltpu.sync_copy(src_hbm.at[layer, pl.ds(base, IG)], sidx)
            pltpu.sync_copy(dst_hbm.at[layer, pl.ds(base, IG)], didx)

            for b in range(NBUF):
                pltpu.async_copy(u_hbm.at[sidx.at[b]], rows.at[b],
                                 gsem.at[b])

            def group_body(g, carry, acc=acc):
                for b in range(NBUF):
                    j = g * NBUF + b
                    pltpu.make_async_copy(
                        u_hbm.at[sidx.at[j]], rows.at[b], gsem.at[b]).wait()
                    pltpu.async_copy(rows.at[b], acc.at[didx.at[j]],
                                     ssem.at[b], add=True)
                for b in range(NBUF):
                    j = g * NBUF + b
                    pltpu.make_async_copy(
                        rows.at[b], acc.at[didx.at[j]], ssem.at[b]).wait()

                    @pl.when(j + NBUF < IG)
                    def _(j=j, b=b):
                        pltpu.async_copy(
                            u_hbm.at[sidx.at[j + NBUF]], rows.at[b],
                            gsem.at[b])
                return carry

            lax.fori_loop(0, IG // NBUF, group_body, 0)

    plsc.subcore_barrier()
    pltpu.sync_copy(acc0.at[pl.ds(t * ZROWS, ZROWS)],
                    p_hbm.at[c * 2, pl.ds(t * ZROWS, ZROWS)])
    pltpu.sync_copy(acc1.at[pl.ds(t * ZROWS, ZROWS)],
                    p_hbm.at[c * 2 + 1, pl.ds(t * ZROWS, ZROWS)])


_hop = functools.partial(
    pl.kernel,
    _hop_body,
    out_type=jax.ShapeDtypeStruct((N_LAYERS, ACC_ROWS, DIM), jnp.float32),
    mesh=plsc.VectorSubcoreMesh(core_axis_name="c", subcore_axis_name="s"),
    compiler_params=pltpu.CompilerParams(use_tc_tiling_on_sc=False),
    scratch_types=[
        pltpu.VMEM((IG, CHUNK), jnp.int32),
        pltpu.VMEM((IG, CHUNK), jnp.int32),
        pltpu.VMEM((NBUF, CHUNK, DIM), jnp.float32),
        pltpu.VMEM_SHARED((ACC_ROWS, DIM), jnp.float32),
        pltpu.VMEM_SHARED((ACC_ROWS, DIM), jnp.float32),
        pltpu.SemaphoreType.DMA((NBUF,)),
        pltpu.SemaphoreType.DMA((NBUF,)),
    ],
)()


def _merge_body(p_ref, wb_ref, o_ref):
    o_ref[...] = (wb_ref[0, 0] * p_ref[0] + wb_ref[1, 0] * p_ref[1]
                  + wb_ref[2, 0] * p_ref[2] + wb_ref[3, 0] * p_ref[3])


def _merge(P, wb):
    blk = 1000
    return pl.pallas_call(
        _merge_body,
        grid=(N // blk,),
        in_specs=[
            pl.BlockSpec((N_LAYERS, blk, DIM), lambda i: (0, i, 0)),
            pl.BlockSpec(memory_space=pltpu.SMEM),
        ],
        out_specs=pl.BlockSpec((blk, DIM), lambda i: (i, 0)),
        out_shape=jax.ShapeDtypeStruct((N, DIM), jnp.float32),
    )(P, wb)


def _merge_final_body(p_ref, wb_ref, wa_ref, u1_ref, u2_ref, o_ref):
    u3 = (wb_ref[0, 0] * p_ref[0] + wb_ref[1, 0] * p_ref[1]
          + wb_ref[2, 0] * p_ref[2] + wb_ref[3, 0] * p_ref[3])
    o_ref[...] = (wa_ref[0, 0] * u1_ref[...] + wa_ref[1, 0] * u2_ref[...]
                  + wa_ref[2, 0] * u3)


def _merge_final(P, wb, wa, U1, U2):
    blk = 1000
    return pl.pallas_call(
        _merge_final_body,
        grid=(N // blk,),
        in_specs=[
            pl.BlockSpec((N_LAYERS, blk, DIM), lambda i: (0, i, 0)),
            pl.BlockSpec(memory_space=pltpu.SMEM),
            pl.BlockSpec(memory_space=pltpu.SMEM),
            pl.BlockSpec((blk, DIM), lambda i: (i, 0)),
            pl.BlockSpec((blk, DIM), lambda i: (i, 0)),
        ],
        out_specs=pl.BlockSpec((blk, DIM), lambda i: (i, 0)),
        out_shape=jax.ShapeDtypeStruct((N, DIM), jnp.float32),
    )(P, wb, wa, U1, U2)


def kernel(feature, edge_index, weight_b, weight_a, G):
    src = edge_index[:, 0, :].reshape(N_LAYERS, CHUNKS_PER_LAYER, CHUNK)
    dst = edge_index[:, 1, :].reshape(N_LAYERS, CHUNKS_PER_LAYER, CHUNK)
    pad = CHUNKS_PAD - CHUNKS_PER_LAYER
    src = jnp.pad(src, ((0, 0), (0, pad), (0, 0)))
    dst = jnp.pad(dst, ((0, 0), (0, pad), (0, 0)), constant_values=N)
    zeros = jnp.zeros((ZROWS, DIM), jnp.float32)

    U = _project(feature, G)
    P1 = _hop(U, src, dst, zeros)
    U1 = _merge(P1, weight_b)
    P2 = _hop(U1, src, dst, zeros)
    U2 = _merge(P2, weight_b)
    P3 = _hop(U2, src, dst, zeros)
    return _merge_final(P3, weight_b, weight_a, U1, U2)


# speedup vs baseline: 2.0902x; 2.0902x over previous
import functools

import jax
import jax.numpy as jnp
from jax import lax
from jax.experimental import pallas as pl
from jax.experimental.pallas import tpu as pltpu
from jax.experimental.pallas import tpu_sc as plsc

N = 10000
D_FEAT = 128
DIM = 64
Q = 3
N_LAYERS = 4
E_PER = 320000

CHUNK = 128
CHUNKS_PER_LAYER = E_PER // CHUNK
NC, NS = 2, 16
CPT = 160
CHUNKS_PAD = CPT * NS
ACC_ROWS = 10240
ZROWS = ACC_ROWS // NS


def _proj_body(f_ref, g_ref, o_ref):
    f = f_ref[...]
    ss = jnp.sum(f * f, axis=1, keepdims=True)
    fn = f / (jnp.sqrt(ss) + 1e-12)
    o_ref[...] = jnp.dot(fn, g_ref[...], preferred_element_type=jnp.float32)


def _project(feature, G):
    blk = 1000
    grid = N // blk
    return pl.pallas_call(
        _proj_body,
        grid=(grid,),
        in_specs=[
            pl.BlockSpec((blk, D_FEAT), lambda i: (i, 0)),
            pl.BlockSpec((D_FEAT, DIM), lambda i: (0, 0)),
        ],
        out_specs=pl.BlockSpec((blk, DIM), lambda i: (i, 0)),
        out_shape=jax.ShapeDtypeStruct((N, DIM), jnp.float32),
    )(feature, G)


NBUF = 4
IG = 32


def _hop_body(u_hbm, src_hbm, dst_hbm, zeros_hbm, p_hbm,
              sidx, didx, rows, ushr, acc, gsem, ssem):
    c = lax.axis_index("c")
    t = lax.axis_index("s")

    pltpu.sync_copy(u_hbm.at[pl.ds(t * ZROWS, ZROWS)],
                    ushr.at[pl.ds(t * ZROWS, ZROWS)])

    for ll in range(2):
        pltpu.sync_copy(zeros_hbm, acc.at[pl.ds(t * ZROWS, ZROWS)])
        plsc.subcore_barrier()
        layer = c * 2 + ll
        for ig in range(CPT // IG):
            base = t * CPT + ig * IG
            pltpu.sync_copy(src_hbm.at[layer, pl.ds(base, IG)], sidx)
            pltpu.sync_copy(dst_hbm.at[layer, pl.ds(base, IG)], didx)

            for b in range(NBUF):
                pltpu.async_copy(ushr.at[sidx.at[b]], rows.at[b],
                                 gsem.at[b])

            def group_body(g, carry):
                for b in range(NBUF):
                    j = g * NBUF + b
                    pltpu.make_async_copy(
                        ushr.at[sidx.at[j]], rows.at[b], gsem.at[b]).wait()
                    pltpu.async_copy(rows.at[b], acc.at[didx.at[j]],
                                     ssem.at[b], add=True)
                for b in range(NBUF):
                    j = g * NBUF + b
                    pltpu.make_async_copy(
                        rows.at[b], acc.at[didx.at[j]], ssem.at[b]).wait()

                    @pl.when(j + NBUF < IG)
                    def _(j=j, b=b):
                        pltpu.async_copy(
                            ushr.at[sidx.at[j + NBUF]], rows.at[b],
                            gsem.at[b])
                return carry

            lax.fori_loop(0, IG // NBUF, group_body, 0)

        plsc.subcore_barrier()
        pltpu.sync_copy(acc.at[pl.ds(t * ZROWS, ZROWS)],
                        p_hbm.at[layer, pl.ds(t * ZROWS, ZROWS)])
        plsc.subcore_barrier()


_hop = functools.partial(
    pl.kernel,
    _hop_body,
    out_type=jax.ShapeDtypeStruct((N_LAYERS, ACC_ROWS, DIM), jnp.float32),
    mesh=plsc.VectorSubcoreMesh(core_axis_name="c", subcore_axis_name="s"),
    compiler_params=pltpu.CompilerParams(use_tc_tiling_on_sc=False),
    scratch_types=[
        pltpu.VMEM((IG, CHUNK), jnp.int32),
        pltpu.VMEM((IG, CHUNK), jnp.int32),
        pltpu.VMEM((NBUF, CHUNK, DIM), jnp.float32),
        pltpu.VMEM_SHARED((ACC_ROWS, DIM), jnp.float32),
        pltpu.VMEM_SHARED((ACC_ROWS, DIM), jnp.float32),
        pltpu.SemaphoreType.DMA((NBUF,)),
        pltpu.SemaphoreType.DMA((NBUF,)),
    ],
)()


def _merge_body(p_ref, wb_ref, o_ref):
    o_ref[...] = (wb_ref[0, 0] * p_ref[0] + wb_ref[1, 0] * p_ref[1]
                  + wb_ref[2, 0] * p_ref[2] + wb_ref[3, 0] * p_ref[3])


def _merge(P, wb):
    blk = 1024
    return pl.pallas_call(
        _merge_body,
        grid=(ACC_ROWS // blk,),
        in_specs=[
            pl.BlockSpec((N_LAYERS, blk, DIM), lambda i: (0, i, 0)),
            pl.BlockSpec(memory_space=pltpu.SMEM),
        ],
        out_specs=pl.BlockSpec((blk, DIM), lambda i: (i, 0)),
        out_shape=jax.ShapeDtypeStruct((ACC_ROWS, DIM), jnp.float32),
    )(P, wb)


def _merge_final_body(p_ref, wb_ref, wa_ref, u1_ref, u2_ref, o_ref):
    u3 = (wb_ref[0, 0] * p_ref[0] + wb_ref[1, 0] * p_ref[1]
          + wb_ref[2, 0] * p_ref[2] + wb_ref[3, 0] * p_ref[3])
    o_ref[...] = (wa_ref[0, 0] * u1_ref[...] + wa_ref[1, 0] * u2_ref[...]
                  + wa_ref[2, 0] * u3)


def _merge_final(P, wb, wa, U1, U2):
    blk = 1000
    return pl.pallas_call(
        _merge_final_body,
        grid=(N // blk,),
        in_specs=[
            pl.BlockSpec((N_LAYERS, blk, DIM), lambda i: (0, i, 0)),
            pl.BlockSpec(memory_space=pltpu.SMEM),
            pl.BlockSpec(memory_space=pltpu.SMEM),
            pl.BlockSpec((blk, DIM), lambda i: (i, 0)),
            pl.BlockSpec((blk, DIM), lambda i: (i, 0)),
        ],
        out_specs=pl.BlockSpec((blk, DIM), lambda i: (i, 0)),
        out_shape=jax.ShapeDtypeStruct((N, DIM), jnp.float32),
    )(P, wb, wa, U1, U2)


def kernel(feature, edge_index, weight_b, weight_a, G):
    src = edge_index[:, 0, :].reshape(N_LAYERS, CHUNKS_PER_LAYER, CHUNK)
    dst = edge_index[:, 1, :].reshape(N_LAYERS, CHUNKS_PER_LAYER, CHUNK)
    pad = CHUNKS_PAD - CHUNKS_PER_LAYER
    src = jnp.pad(src, ((0, 0), (0, pad), (0, 0)))
    dst = jnp.pad(dst, ((0, 0), (0, pad), (0, 0)), constant_values=N)
    zeros = jnp.zeros((ZROWS, DIM), jnp.float32)

    U = jnp.pad(_project(feature, G), ((0, ACC_ROWS - N), (0, 0)))
    P1 = _hop(U, src, dst, zeros)
    U1 = _merge(P1, weight_b)
    P2 = _hop(U1, src, dst, zeros)
    U2 = _merge(P2, weight_b)
    P3 = _hop(U2, src, dst, zeros)
    return _merge_final(P3, weight_b, weight_a, U1, U2)


# speedup vs baseline: 3.2939x; 1.5759x over previous
import functools

import jax
import jax.numpy as jnp
from jax import lax
from jax.experimental import pallas as pl
from jax.experimental.pallas import tpu as pltpu
from jax.experimental.pallas import tpu_sc as plsc

N = 10000
D_FEAT = 128
DIM = 64
Q = 3
N_LAYERS = 4
E_PER = 320000

CHUNK = 128
CHUNKS_PER_LAYER = E_PER // CHUNK
NC, NS = 2, 16
CPT = 160
CHUNKS_PAD = CPT * NS
ACC_ROWS = 10240
ZROWS = ACC_ROWS // NS


def _proj_body(f_ref, g_ref, o_ref):
    f = f_ref[...]
    ss = jnp.sum(f * f, axis=1, keepdims=True)
    fn = f / (jnp.sqrt(ss) + 1e-12)
    o_ref[...] = jnp.dot(
        fn, g_ref[...], preferred_element_type=jnp.float32
    ).astype(jnp.bfloat16)


def _project(feature, G):
    blk = 2000
    grid = N // blk
    return pl.pallas_call(
        _proj_body,
        grid=(grid,),
        in_specs=[
            pl.BlockSpec((blk, D_FEAT), lambda i: (i, 0)),
            pl.BlockSpec((D_FEAT, DIM), lambda i: (0, 0)),
        ],
        out_specs=pl.BlockSpec((blk, DIM), lambda i: (i, 0)),
        out_shape=jax.ShapeDtypeStruct((N, DIM), jnp.bfloat16),
    )(feature, G)


NBUF = 4
IG = 32


def _hop_body(u_hbm, src_hbm, dst_hbm, zeros_hbm, p_hbm,
              sidx, didx, rows, ushr, acc0, acc1, gsem, ssem):
    c = lax.axis_index("c")
    t = lax.axis_index("s")

    pltpu.sync_copy(u_hbm.at[pl.ds(t * ZROWS, ZROWS)],
                    ushr.at[pl.ds(t * ZROWS, ZROWS)])
    pltpu.sync_copy(zeros_hbm, acc0.at[pl.ds(t * ZROWS, ZROWS)])
    pltpu.sync_copy(zeros_hbm, acc1.at[pl.ds(t * ZROWS, ZROWS)])
    plsc.subcore_barrier()

    for ll in range(2):
        acc = acc0 if ll == 0 else acc1
        layer = c * 2 + ll
        for ig in range(CPT // IG):
            base = t * CPT + ig * IG
            pltpu.sync_copy(src_hbm.at[layer, pl.ds(base, IG)], sidx)
            pltpu.sync_copy(dst_hbm.at[layer, pl.ds(base, IG)], didx)

            for b in range(NBUF):
                pltpu.async_copy(ushr.at[sidx.at[b]], rows.at[b],
                                 gsem.at[b])

            def group_body(g, carry, acc=acc):
                for b in range(NBUF):
                    j = g * NBUF + b
                    pltpu.make_async_copy(
                        ushr.at[sidx.at[j]], rows.at[b], gsem.at[b]).wait()
                    pltpu.async_copy(rows.at[b], acc.at[didx.at[j]],
                                     ssem.at[b], add=True)
                for b in range(NBUF):
                    j = g * NBUF + b
                    pltpu.make_async_copy(
                        rows.at[b], acc.at[didx.at[j]], ssem.at[b]).wait()

                    @pl.when(j + NBUF < IG)
                    def _(j=j, b=b):
                        pltpu.async_copy(
                            ushr.at[sidx.at[j + NBUF]], rows.at[b],
                            gsem.at[b])
                return carry

            lax.fori_loop(0, IG // NBUF, group_body, 0)

    plsc.subcore_barrier()
    pltpu.sync_copy(acc0.at[pl.ds(t * ZROWS, ZROWS)],
                    p_hbm.at[c * 2, pl.ds(t * ZROWS, ZROWS)])
    pltpu.sync_copy(acc1.at[pl.ds(t * ZROWS, ZROWS)],
                    p_hbm.at[c * 2 + 1, pl.ds(t * ZROWS, ZROWS)])


_hop = functools.partial(
    pl.kernel,
    _hop_body,
    out_type=jax.ShapeDtypeStruct((N_LAYERS, ACC_ROWS, DIM), jnp.bfloat16),
    mesh=plsc.VectorSubcoreMesh(core_axis_name="c", subcore_axis_name="s"),
    compiler_params=pltpu.CompilerParams(use_tc_tiling_on_sc=False),
    scratch_types=[
        pltpu.VMEM((IG, CHUNK), jnp.int32),
        pltpu.VMEM((IG, CHUNK), jnp.int32),
        pltpu.VMEM((NBUF, CHUNK, DIM), jnp.bfloat16),
        pltpu.VMEM_SHARED((ACC_ROWS, DIM), jnp.bfloat16),
        pltpu.VMEM_SHARED((ACC_ROWS, DIM), jnp.bfloat16),
        pltpu.VMEM_SHARED((ACC_ROWS, DIM), jnp.bfloat16),
        pltpu.SemaphoreType.DMA((NBUF,)),
        pltpu.SemaphoreType.DMA((NBUF,)),
    ],
)()


def _merge_body(p_ref, wb_ref, o_ref):
    p = p_ref[...].astype(jnp.float32)
    o_ref[...] = (wb_ref[0, 0] * p[0] + wb_ref[1, 0] * p[1]
                  + wb_ref[2, 0] * p[2]
                  + wb_ref[3, 0] * p[3]).astype(jnp.bfloat16)


def _merge(P, wb):
    blk = 1024
    return pl.pallas_call(
        _merge_body,
        grid=(ACC_ROWS // blk,),
        in_specs=[
            pl.BlockSpec((N_LAYERS, blk, DIM), lambda i: (0, i, 0)),
            pl.BlockSpec(memory_space=pltpu.SMEM),
        ],
        out_specs=pl.BlockSpec((blk, DIM), lambda i: (i, 0)),
        out_shape=jax.ShapeDtypeStruct((ACC_ROWS, DIM), jnp.bfloat16),
    )(P, wb)


def _merge_final_body(p_ref, wb_ref, wa_ref, u1_ref, u2_ref, o_ref):
    p = p_ref[...].astype(jnp.float32)
    u3 = (wb_ref[0, 0] * p[0] + wb_ref[1, 0] * p[1]
          + wb_ref[2, 0] * p[2] + wb_ref[3, 0] * p[3])
    o_ref[...] = (wa_ref[0, 0] * u1_ref[...].astype(jnp.float32)
                  + wa_ref[1, 0] * u2_ref[...].astype(jnp.float32)
                  + wa_ref[2, 0] * u3)


def _merge_final(P, wb, wa, U1, U2):
    blk = 2000
    return pl.pallas_call(
        _merge_final_body,
        grid=(N // blk,),
        in_specs=[
            pl.BlockSpec((N_LAYERS, blk, DIM), lambda i: (0, i, 0)),
            pl.BlockSpec(memory_space=pltpu.SMEM),
            pl.BlockSpec(memory_space=pltpu.SMEM),
            pl.BlockSpec((blk, DIM), lambda i: (i, 0)),
            pl.BlockSpec((blk, DIM), lambda i: (i, 0)),
        ],
        out_specs=pl.BlockSpec((blk, DIM), lambda i: (i, 0)),
        out_shape=jax.ShapeDtypeStruct((N, DIM), jnp.float32),
    )(P, wb, wa, U1, U2)


def kernel(feature, edge_index, weight_b, weight_a, G):
    src = edge_index[:, 0, :].reshape(N_LAYERS, CHUNKS_PER_LAYER, CHUNK)
    dst = edge_index[:, 1, :].reshape(N_LAYERS, CHUNKS_PER_LAYER, CHUNK)
    pad = CHUNKS_PAD - CHUNKS_PER_LAYER
    src = jnp.pad(src, ((0, 0), (0, pad), (0, 0)))
    dst = jnp.pad(dst, ((0, 0), (0, pad), (0, 0)), constant_values=N)
    zeros = jnp.zeros((ZROWS, DIM), jnp.bfloat16)

    U = jnp.pad(_project(feature, G), ((0, ACC_ROWS - N), (0, 0)))
    P1 = _hop(U, src, dst, zeros)
    U1 = _merge(P1, weight_b)
    P2 = _hop(U1, src, dst, zeros)
    U2 = _merge(P2, weight_b)
    P3 = _hop(U2, src, dst, zeros)
    return _merge_final(P3, weight_b, weight_a, U1, U2)


# speedup vs baseline: 3.5174x; 1.0678x over previous
import functools

import jax
import jax.numpy as jnp
from jax import lax
from jax.experimental import pallas as pl
from jax.experimental.pallas import tpu as pltpu
from jax.experimental.pallas import tpu_sc as plsc

N = 10000
D_FEAT = 128
DIM = 64
Q = 3
N_LAYERS = 4
E_PER = 320000

CHUNK = 128
CHUNKS_PER_LAYER = E_PER // CHUNK
NC, NS = 2, 16
CPT = 160
CHUNKS_PAD = CPT * NS
ACC_ROWS = 10240
ZROWS = ACC_ROWS // NS


def _proj_body(f_ref, g_ref, o_ref):
    f = f_ref[...]
    ss = jnp.sum(f * f, axis=1, keepdims=True)
    fn = f / (jnp.sqrt(ss) + 1e-12)
    o_ref[...] = jnp.dot(
        fn, g_ref[...], preferred_element_type=jnp.float32
    ).astype(jnp.bfloat16)


def _project(feature, G):
    blk = 2000
    grid = N // blk
    return pl.pallas_call(
        _proj_body,
        grid=(grid,),
        in_specs=[
            pl.BlockSpec((blk, D_FEAT), lambda i: (i, 0)),
            pl.BlockSpec((D_FEAT, DIM), lambda i: (0, 0)),
        ],
        out_specs=pl.BlockSpec((blk, DIM), lambda i: (i, 0)),
        out_shape=jax.ShapeDtypeStruct((N, DIM), jnp.bfloat16),
    )(feature, G)


NBUF = 8
IG = 32


def _hop_body(u_hbm, src_hbm, dst_hbm, zeros_hbm, p_hbm,
              sidx, didx, rows, ushr, acc0, acc1, gsem, ssem):
    c = lax.axis_index("c")
    t = lax.axis_index("s")

    pltpu.sync_copy(u_hbm.at[pl.ds(t * ZROWS, ZROWS)],
                    ushr.at[pl.ds(t * ZROWS, ZROWS)])
    pltpu.sync_copy(zeros_hbm, acc0.at[pl.ds(t * ZROWS, ZROWS)])
    pltpu.sync_copy(zeros_hbm, acc1.at[pl.ds(t * ZROWS, ZROWS)])
    plsc.subcore_barrier()

    for ll in range(2):
        acc = acc0 if ll == 0 else acc1
        layer = c * 2 + ll
        for ig in range(CPT // IG):
            base = t * CPT + ig * IG
            pltpu.sync_copy(src_hbm.at[layer, pl.ds(base, IG)], sidx)
            pltpu.sync_copy(dst_hbm.at[layer, pl.ds(base, IG)], didx)

            for b in range(NBUF):
                pltpu.async_copy(ushr.at[sidx.at[b]], rows.at[b],
                                 gsem.at[b])

            def group_body(g, carry, acc=acc):
                for b in range(NBUF):
                    j = g * NBUF + b
                    pltpu.make_async_copy(
                        ushr.at[sidx.at[j]], rows.at[b], gsem.at[b]).wait()
                    pltpu.async_copy(rows.at[b], acc.at[didx.at[j]],
                                     ssem.at[b], add=True)
                for b in range(NBUF):
                    j = g * NBUF + b
                    pltpu.make_async_copy(
                        rows.at[b], acc.at[didx.at[j]], ssem.at[b]).wait()

                    @pl.when(j + NBUF < IG)
                    def _(j=j, b=b):
                        pltpu.async_copy(
                            ushr.at[sidx.at[j + NBUF]], rows.at[b],
                            gsem.at[b])
                return carry

            lax.fori_loop(0, IG // NBUF, group_body, 0)

    plsc.subcore_barrier()
    pltpu.sync_copy(acc0.at[pl.ds(t * ZROWS, ZROWS)],
                    p_hbm.at[c * 2, pl.ds(t * ZROWS, ZROWS)])
    pltpu.sync_copy(acc1.at[pl.ds(t * ZROWS, ZROWS)],
                    p_hbm.at[c * 2 + 1, pl.ds(t * ZROWS, ZROWS)])


_hop = functools.partial(
    pl.kernel,
    _hop_body,
    out_type=jax.ShapeDtypeStruct((N_LAYERS, ACC_ROWS, DIM), jnp.bfloat16),
    mesh=plsc.VectorSubcoreMesh(core_axis_name="c", subcore_axis_name="s"),
    compiler_params=pltpu.CompilerParams(use_tc_tiling_on_sc=False),
    scratch_types=[
        pltpu.VMEM((IG, CHUNK), jnp.int32),
        pltpu.VMEM((IG, CHUNK), jnp.int32),
        pltpu.VMEM((NBUF, CHUNK, DIM), jnp.bfloat16),
        pltpu.VMEM_SHARED((ACC_ROWS, DIM), jnp.bfloat16),
        pltpu.VMEM_SHARED((ACC_ROWS, DIM), jnp.bfloat16),
        pltpu.VMEM_SHARED((ACC_ROWS, DIM), jnp.bfloat16),
        pltpu.SemaphoreType.DMA((NBUF,)),
        pltpu.SemaphoreType.DMA((NBUF,)),
    ],
)()


def _merge_body(p_ref, wb_ref, o_ref):
    p = p_ref[...].astype(jnp.float32)
    o_ref[...] = (wb_ref[0, 0] * p[0] + wb_ref[1, 0] * p[1]
                  + wb_ref[2, 0] * p[2]
                  + wb_ref[3, 0] * p[3]).astype(jnp.bfloat16)


def _merge(P, wb):
    blk = 1024
    return pl.pallas_call(
        _merge_body,
        grid=(ACC_ROWS // blk,),
        in_specs=[
            pl.BlockSpec((N_LAYERS, blk, DIM), lambda i: (0, i, 0)),
            pl.BlockSpec(memory_space=pltpu.SMEM),
        ],
        out_specs=pl.BlockSpec((blk, DIM), lambda i: (i, 0)),
        out_shape=jax.ShapeDtypeStruct((ACC_ROWS, DIM), jnp.bfloat16),
    )(P, wb)


def _merge_final_body(p_ref, wb_ref, wa_ref, u1_ref, u2_ref, o_ref):
    p = p_ref[...].astype(jnp.float32)
    u3 = (wb_ref[0, 0] * p[0] + wb_ref[1, 0] * p[1]
          + wb_ref[2, 0] * p[2] + wb_ref[3, 0] * p[3])
    o_ref[...] = (wa_ref[0, 0] * u1_ref[...].astype(jnp.float32)
                  + wa_ref[1, 0] * u2_ref[...].astype(jnp.float32)
                  + wa_ref[2, 0] * u3)


def _merge_final(P, wb, wa, U1, U2):
    blk = 2000
    return pl.pallas_call(
        _merge_final_body,
        grid=(N // blk,),
        in_specs=[
            pl.BlockSpec((N_LAYERS, blk, DIM), lambda i: (0, i, 0)),
            pl.BlockSpec(memory_space=pltpu.SMEM),
            pl.BlockSpec(memory_space=pltpu.SMEM),
            pl.BlockSpec((blk, DIM), lambda i: (i, 0)),
            pl.BlockSpec((blk, DIM), lambda i: (i, 0)),
        ],
        out_specs=pl.BlockSpec((blk, DIM), lambda i: (i, 0)),
        out_shape=jax.ShapeDtypeStruct((N, DIM), jnp.float32),
    )(P, wb, wa, U1, U2)


def kernel(feature, edge_index, weight_b, weight_a, G):
    src = edge_index[:, 0, :].reshape(N_LAYERS, CHUNKS_PER_LAYER, CHUNK)
    dst = edge_index[:, 1, :].reshape(N_LAYERS, CHUNKS_PER_LAYER, CHUNK)
    pad = CHUNKS_PAD - CHUNKS_PER_LAYER
    src = jnp.pad(src, ((0, 0), (0, pad), (0, 0)))
    dst = jnp.pad(dst, ((0, 0), (0, pad), (0, 0)), constant_values=N)
    zeros = jnp.zeros((ZROWS, DIM), jnp.bfloat16)

    U = jnp.pad(_project(feature, G), ((0, ACC_ROWS - N), (0, 0)))
    P1 = _hop(U, src, dst, zeros)
    U1 = _merge(P1, weight_b)
    P2 = _hop(U1, src, dst, zeros)
    U2 = _merge(P2, weight_b)
    P3 = _hop(U2, src, dst, zeros)
    return _merge_final(P3, weight_b, weight_a, U1, U2)


# speedup vs baseline: 3.6826x; 1.0470x over previous
import functools

import jax
import jax.numpy as jnp
from jax import lax
from jax.experimental import pallas as pl
from jax.experimental.pallas import tpu as pltpu
from jax.experimental.pallas import tpu_sc as plsc

N = 10000
D_FEAT = 128
DIM = 64
Q = 3
N_LAYERS = 4
E_PER = 320000

CHUNK = 128
CHUNKS_PER_LAYER = E_PER // CHUNK
NC, NS = 2, 16
CPT = 160
CHUNKS_PAD = CPT * NS
ACC_ROWS = 10240
ZROWS = ACC_ROWS // NS


def _proj_body(f_ref, g_ref, o_ref):
    f = f_ref[...]
    ss = jnp.sum(f * f, axis=1, keepdims=True)
    fn = f / (jnp.sqrt(ss) + 1e-12)
    o_ref[...] = jnp.dot(
        fn, g_ref[...], preferred_element_type=jnp.float32
    ).astype(jnp.bfloat16)


def _project(feature, G):
    blk = 2000
    grid = N // blk
    return pl.pallas_call(
        _proj_body,
        grid=(grid,),
        in_specs=[
            pl.BlockSpec((blk, D_FEAT), lambda i: (i, 0)),
            pl.BlockSpec((D_FEAT, DIM), lambda i: (0, 0)),
        ],
        out_specs=pl.BlockSpec((blk, DIM), lambda i: (i, 0)),
        out_shape=jax.ShapeDtypeStruct((N, DIM), jnp.bfloat16),
    )(feature, G)


NBUF = 8
IG = 32


def _hop_body(u_hbm, src_hbm, dst_hbm, zeros_hbm, p_hbm,
              sidx, didx, rows, ushr, acc0, acc1, gsem, ssem):
    c = lax.axis_index("c")
    t = lax.axis_index("s")

    pltpu.sync_copy(u_hbm.at[pl.ds(t * ZROWS, ZROWS)],
                    ushr.at[pl.ds(t * ZROWS, ZROWS)])
    pltpu.sync_copy(zeros_hbm, acc0.at[pl.ds(t * ZROWS, ZROWS)])
    pltpu.sync_copy(zeros_hbm, acc1.at[pl.ds(t * ZROWS, ZROWS)])
    plsc.subcore_barrier()

    for ll in range(2):
        acc = acc0 if ll == 0 else acc1
        layer = c * 2 + ll
        for ig in range(CPT // IG):
            base = t * CPT + ig * IG
            pltpu.sync_copy(src_hbm.at[layer, pl.ds(base, IG)], sidx)
            pltpu.sync_copy(dst_hbm.at[layer, pl.ds(base, IG)], didx)

            for b in range(NBUF):
                pltpu.async_copy(ushr.at[sidx.at[b]], rows.at[b],
                                 gsem.at[b])

            def group_body(g, carry, acc=acc):
                for b in range(NBUF):
                    j = g * NBUF + b
                    pltpu.make_async_copy(
                        ushr.at[sidx.at[j]], rows.at[b], gsem.at[b]).wait()
                    pltpu.async_copy(rows.at[b], acc.at[didx.at[j]],
                                     ssem.at[b], add=True)
                for b in range(NBUF):
                    j = g * NBUF + b
                    pltpu.make_async_copy(
                        rows.at[b], acc.at[didx.at[j]], ssem.at[b]).wait()

                    @pl.when(j + NBUF < IG)
                    def _(j=j, b=b):
                        pltpu.async_copy(
                            ushr.at[sidx.at[j + NBUF]], rows.at[b],
                            gsem.at[b])
                return carry

            lax.fori_loop(0, IG // NBUF, group_body, 0)

    plsc.subcore_barrier()
    pltpu.sync_copy(acc0.at[pl.ds(t * ZROWS, ZROWS)],
                    p_hbm.at[c * 2, pl.ds(t * ZROWS, ZROWS)])
    pltpu.sync_copy(acc1.at[pl.ds(t * ZROWS, ZROWS)],
                    p_hbm.at[c * 2 + 1, pl.ds(t * ZROWS, ZROWS)])


_hop = functools.partial(
    pl.kernel,
    _hop_body,
    out_type=jax.ShapeDtypeStruct((N_LAYERS, ACC_ROWS, DIM), jnp.bfloat16),
    mesh=plsc.VectorSubcoreMesh(core_axis_name="c", subcore_axis_name="s"),
    compiler_params=pltpu.CompilerParams(use_tc_tiling_on_sc=False),
    scratch_types=[
        pltpu.VMEM((IG, CHUNK), jnp.int32),
        pltpu.VMEM((IG, CHUNK), jnp.int32),
        pltpu.VMEM((NBUF, CHUNK, DIM), jnp.bfloat16),
        pltpu.VMEM_SHARED((ACC_ROWS, DIM), jnp.bfloat16),
        pltpu.VMEM_SHARED((ACC_ROWS, DIM), jnp.bfloat16),
        pltpu.VMEM_SHARED((ACC_ROWS, DIM), jnp.bfloat16),
        pltpu.SemaphoreType.DMA((NBUF,)),
        pltpu.SemaphoreType.DMA((NBUF,)),
    ],
)()


MROWS = 160
VPM = MROWS * DIM // 32


def _hop_m_body(p_prev_hbm, wbv_hbm, src_hbm, dst_hbm, zeros_hbm,
                p_hbm, u_hbm,
                sidx, didx, rows, pbuf, ubuf, wbuf,
                ushr, acc0, acc1, gsem, ssem):
    c = lax.axis_index("c")
    t = lax.axis_index("s")

    pltpu.sync_copy(wbv_hbm, wbuf)
    for m in range(ZROWS // MROWS):
        r0 = t * ZROWS + m * MROWS
        for l in range(N_LAYERS):
            pltpu.sync_copy(p_prev_hbm.at[l, pl.ds(r0, MROWS)],
                            pbuf.at[l])

        def merge_row(r, carry):
            for c2 in range(0, DIM, 32):
                x = wbuf[0] * pbuf[0, r, pl.ds(c2, 32)]
                x = x + wbuf[1] * pbuf[1, r, pl.ds(c2, 32)]
                x = x + wbuf[2] * pbuf[2, r, pl.ds(c2, 32)]
                x = x + wbuf[3] * pbuf[3, r, pl.ds(c2, 32)]
                ubuf[r, pl.ds(c2, 32)] = x
            return carry

        lax.fori_loop(0, MROWS, merge_row, 0)
        pltpu.sync_copy(ubuf, ushr.at[pl.ds(r0, MROWS)])

        @pl.when(c == 0)
        def _(r0=r0):
            pltpu.sync_copy(ubuf, u_hbm.at[pl.ds(r0, MROWS)])

    pltpu.sync_copy(zeros_hbm, acc0.at[pl.ds(t * ZROWS, ZROWS)])
    pltpu.sync_copy(zeros_hbm, acc1.at[pl.ds(t * ZROWS, ZROWS)])
    plsc.subcore_barrier()

    for ll in range(2):
        acc = acc0 if ll == 0 else acc1
        layer = c * 2 + ll
        for ig in range(CPT // IG):
            base = t * CPT + ig * IG
            pltpu.sync_copy(src_hbm.at[layer, pl.ds(base, IG)], sidx)
            pltpu.sync_copy(dst_hbm.at[layer, pl.ds(base, IG)], didx)

            for b in range(NBUF):
                pltpu.async_copy(ushr.at[sidx.at[b]], rows.at[b],
                                 gsem.at[b])

            def group_body(g, carry, acc=acc):
                for b in range(NBUF):
                    j = g * NBUF + b
                    pltpu.make_async_copy(
                        ushr.at[sidx.at[j]], rows.at[b], gsem.at[b]).wait()
                    pltpu.async_copy(rows.at[b], acc.at[didx.at[j]],
                                     ssem.at[b], add=True)
                for b in range(NBUF):
                    j = g * NBUF + b
                    pltpu.make_async_copy(
                        rows.at[b], acc.at[didx.at[j]], ssem.at[b]).wait()

                    @pl.when(j + NBUF < IG)
                    def _(j=j, b=b):
                        pltpu.async_copy(
                            ushr.at[sidx.at[j + NBUF]], rows.at[b],
                            gsem.at[b])
                return carry

            lax.fori_loop(0, IG // NBUF, group_body, 0)

    plsc.subcore_barrier()
    pltpu.sync_copy(acc0.at[pl.ds(t * ZROWS, ZROWS)],
                    p_hbm.at[c * 2, pl.ds(t * ZROWS, ZROWS)])
    pltpu.sync_copy(acc1.at[pl.ds(t * ZROWS, ZROWS)],
                    p_hbm.at[c * 2 + 1, pl.ds(t * ZROWS, ZROWS)])


_hop_m = functools.partial(
    pl.kernel,
    _hop_m_body,
    out_type=(
        jax.ShapeDtypeStruct((N_LAYERS, ACC_ROWS, DIM), jnp.bfloat16),
        jax.ShapeDtypeStruct((ACC_ROWS, DIM), jnp.bfloat16),
    ),
    mesh=plsc.VectorSubcoreMesh(core_axis_name="c", subcore_axis_name="s"),
    compiler_params=pltpu.CompilerParams(use_tc_tiling_on_sc=False),
    scratch_types=[
        pltpu.VMEM((IG, CHUNK), jnp.int32),
        pltpu.VMEM((IG, CHUNK), jnp.int32),
        pltpu.VMEM((NBUF, CHUNK, DIM), jnp.bfloat16),
        pltpu.VMEM((N_LAYERS, MROWS, DIM), jnp.bfloat16),
        pltpu.VMEM((MROWS, DIM), jnp.bfloat16),
        pltpu.VMEM((N_LAYERS, 32), jnp.bfloat16),
        pltpu.VMEM_SHARED((ACC_ROWS, DIM), jnp.bfloat16),
        pltpu.VMEM_SHARED((ACC_ROWS, DIM), jnp.bfloat16),
        pltpu.VMEM_SHARED((ACC_ROWS, DIM), jnp.bfloat16),
        pltpu.SemaphoreType.DMA((NBUF,)),
        pltpu.SemaphoreType.DMA((NBUF,)),
    ],
)()


def _merge_body(p_ref, wb_ref, o_ref):
    p = p_ref[...].astype(jnp.float32)
    o_ref[...] = (wb_ref[0, 0] * p[0] + wb_ref[1, 0] * p[1]
                  + wb_ref[2, 0] * p[2]
                  + wb_ref[3, 0] * p[3]).astype(jnp.bfloat16)


def _merge(P, wb):
    blk = 1024
    return pl.pallas_call(
        _merge_body,
        grid=(ACC_ROWS // blk,),
        in_specs=[
            pl.BlockSpec((N_LAYERS, blk, DIM), lambda i: (0, i, 0)),
            pl.BlockSpec(memory_space=pltpu.SMEM),
        ],
        out_specs=pl.BlockSpec((blk, DIM), lambda i: (i, 0)),
        out_shape=jax.ShapeDtypeStruct((ACC_ROWS, DIM), jnp.bfloat16),
    )(P, wb)


def _merge_final_body(p_ref, wb_ref, wa_ref, u1_ref, u2_ref, o_ref):
    p = p_ref[...].astype(jnp.float32)
    u3 = (wb_ref[0, 0] * p[0] + wb_ref[1, 0] * p[1]
          + wb_ref[2, 0] * p[2] + wb_ref[3, 0] * p[3])
    o_ref[...] = (wa_ref[0, 0] * u1_ref[...].astype(jnp.float32)
                  + wa_ref[1, 0] * u2_ref[...].astype(jnp.float32)
                  + wa_ref[2, 0] * u3)


def _merge_final(P, wb, wa, U1, U2):
    blk = 2000
    return pl.pallas_call(
        _merge_final_body,
        grid=(N // blk,),
        in_specs=[
            pl.BlockSpec((N_LAYERS, blk, DIM), lambda i: (0, i, 0)),
            pl.BlockSpec(memory_space=pltpu.SMEM),
            pl.BlockSpec(memory_space=pltpu.SMEM),
            pl.BlockSpec((blk, DIM), lambda i: (i, 0)),
            pl.BlockSpec((blk, DIM), lambda i: (i, 0)),
        ],
        out_specs=pl.BlockSpec((blk, DIM), lambda i: (i, 0)),
        out_shape=jax.ShapeDtypeStruct((N, DIM), jnp.float32),
    )(P, wb, wa, U1, U2)


def kernel(feature, edge_index, weight_b, weight_a, G):
    src = edge_index[:, 0, :].reshape(N_LAYERS, CHUNKS_PER_LAYER, CHUNK)
    dst = edge_index[:, 1, :].reshape(N_LAYERS, CHUNKS_PER_LAYER, CHUNK)
    pad = CHUNKS_PAD - CHUNKS_PER_LAYER
    src = jnp.pad(src, ((0, 0), (0, pad), (0, 0)))
    dst = jnp.pad(dst, ((0, 0), (0, pad), (0, 0)), constant_values=N)
    zeros = jnp.zeros((ZROWS, DIM), jnp.bfloat16)

    wbv = jnp.broadcast_to(
        weight_b.astype(jnp.bfloat16).reshape(N_LAYERS, 1), (N_LAYERS, 32))

    U = jnp.pad(_project(feature, G), ((0, ACC_ROWS - N), (0, 0)))
    P1 = _hop(U, src, dst, zeros)
    P2, U1 = _hop_m(P1, wbv, src, dst, zeros)
    P3, U2 = _hop_m(P2, wbv, src, dst, zeros)
    return _merge_final(P3, weight_b, weight_a, U1, U2)


# speedup vs baseline: 3.9082x; 1.0613x over previous
import functools

import jax
import jax.numpy as jnp
from jax import lax
from jax.experimental import pallas as pl
from jax.experimental.pallas import tpu as pltpu
from jax.experimental.pallas import tpu_sc as plsc

N = 10000
D_FEAT = 128
DIM = 64
Q = 3
N_LAYERS = 4
E_PER = 320000

CHUNK = 128
CHUNKS_PER_LAYER = E_PER // CHUNK
NC, NS = 2, 16
CPT = 160
CHUNKS_PAD = CPT * NS
ACC_ROWS = 10240
ZROWS = ACC_ROWS // NS


def _proj_body(f_ref, g_ref, o_ref):
    f = f_ref[...]
    ss = jnp.sum(f * f, axis=1, keepdims=True)
    fn = f / (jnp.sqrt(ss) + 1e-12)
    o_ref[...] = jnp.dot(
        fn, g_ref[...], preferred_element_type=jnp.float32
    ).astype(jnp.bfloat16)


def _project(feature, G):
    blk = 2000
    grid = N // blk
    return pl.pallas_call(
        _proj_body,
        grid=(grid,),
        in_specs=[
            pl.BlockSpec((blk, D_FEAT), lambda i: (i, 0)),
            pl.BlockSpec((D_FEAT, DIM), lambda i: (0, 0)),
        ],
        out_specs=pl.BlockSpec((blk, DIM), lambda i: (i, 0)),
        out_shape=jax.ShapeDtypeStruct((N, DIM), jnp.bfloat16),
    )(feature, G)


NBUF = 8
IG = 32


def _hop_body(u_hbm, src_hbm, dst_hbm, zeros_hbm, p_hbm,
              sidx, didx, rows, ushr, acc0, acc1, gsem, ssem):
    c = lax.axis_index("c")
    t = lax.axis_index("s")

    cs = pltpu.async_copy(u_hbm.at[pl.ds(t * ZROWS, ZROWS)],
                          ushr.at[pl.ds(t * ZROWS, ZROWS)], gsem.at[0])
    c0 = pltpu.async_copy(zeros_hbm, acc0.at[pl.ds(t * ZROWS, ZROWS)],
                          gsem.at[1])
    c1 = pltpu.async_copy(zeros_hbm, acc1.at[pl.ds(t * ZROWS, ZROWS)],
                          gsem.at[2])
    cs.wait()
    c0.wait()
    c1.wait()
    plsc.subcore_barrier()

    for ll in range(2):
        acc = acc0 if ll == 0 else acc1
        layer = c * 2 + ll
        for ig in range(CPT // IG):
            base = t * CPT + ig * IG
            ci0 = pltpu.async_copy(src_hbm.at[layer, pl.ds(base, IG)],
                                   sidx, gsem.at[0])
            ci1 = pltpu.async_copy(dst_hbm.at[layer, pl.ds(base, IG)],
                                   didx, gsem.at[1])
            ci0.wait()
            ci1.wait()

            for b in range(NBUF):
                pltpu.async_copy(ushr.at[sidx.at[b]], rows.at[b],
                                 gsem.at[b])

            def group_body(g, carry, acc=acc):
                for b in range(NBUF):
                    j = g * NBUF + b
                    pltpu.make_async_copy(
                        ushr.at[sidx.at[j]], rows.at[b], gsem.at[b]).wait()
                    pltpu.async_copy(rows.at[b], acc.at[didx.at[j]],
                                     ssem.at[b], add=True)
                for b in range(NBUF):
                    j = g * NBUF + b
                    pltpu.make_async_copy(
                        rows.at[b], acc.at[didx.at[j]], ssem.at[b]).wait()

                    @pl.when(j + NBUF < IG)
                    def _(j=j, b=b):
                        pltpu.async_copy(
                            ushr.at[sidx.at[j + NBUF]], rows.at[b],
                            gsem.at[b])
                return carry

            lax.fori_loop(0, IG // NBUF, group_body, 0)

    plsc.subcore_barrier()
    cd0 = pltpu.async_copy(acc0.at[pl.ds(t * ZROWS, ZROWS)],
                           p_hbm.at[c * 2, pl.ds(t * ZROWS, ZROWS)],
                           gsem.at[0])
    cd1 = pltpu.async_copy(acc1.at[pl.ds(t * ZROWS, ZROWS)],
                           p_hbm.at[c * 2 + 1, pl.ds(t * ZROWS, ZROWS)],
                           gsem.at[1])
    cd0.wait()
    cd1.wait()


_hop = functools.partial(
    pl.kernel,
    _hop_body,
    out_type=jax.ShapeDtypeStruct((N_LAYERS, ACC_ROWS, DIM), jnp.bfloat16),
    mesh=plsc.VectorSubcoreMesh(core_axis_name="c", subcore_axis_name="s"),
    compiler_params=pltpu.CompilerParams(use_tc_tiling_on_sc=False),
    scratch_types=[
        pltpu.VMEM((IG, CHUNK), jnp.int32),
        pltpu.VMEM((IG, CHUNK), jnp.int32),
        pltpu.VMEM((NBUF, CHUNK, DIM), jnp.bfloat16),
        pltpu.VMEM_SHARED((ACC_ROWS, DIM), jnp.bfloat16),
        pltpu.VMEM_SHARED((ACC_ROWS, DIM), jnp.bfloat16),
        pltpu.VMEM_SHARED((ACC_ROWS, DIM), jnp.bfloat16),
        pltpu.SemaphoreType.DMA((NBUF,)),
        pltpu.SemaphoreType.DMA((NBUF,)),
    ],
)()


MROWS = 160
VPM = MROWS * DIM // 32


def _hop_m_body(p_prev_hbm, wbv_hbm, src_hbm, dst_hbm, zeros_hbm,
                p_hbm, u_hbm,
                sidx, didx, rows, pbuf, ubuf, wbuf,
                ushr, acc0, acc1, gsem, ssem):
    c = lax.axis_index("c")
    t = lax.axis_index("s")

    cz0 = pltpu.async_copy(zeros_hbm, acc0.at[pl.ds(t * ZROWS, ZROWS)],
                           gsem.at[6])
    cz1 = pltpu.async_copy(zeros_hbm, acc1.at[pl.ds(t * ZROWS, ZROWS)],
                           gsem.at[7])
    pltpu.sync_copy(wbv_hbm, wbuf)

    nm = ZROWS // MROWS
    for m in range(nm):
        r0 = t * ZROWS + m * MROWS
        loads = [
            pltpu.async_copy(p_prev_hbm.at[l, pl.ds(r0, MROWS)],
                             pbuf.at[l], gsem.at[l])
            for l in range(N_LAYERS)
        ]
        if m > 0:
            r0p = t * ZROWS + (m - 1) * MROWS
            pltpu.make_async_copy(
                ubuf, ushr.at[pl.ds(r0p, MROWS)], ssem.at[0]).wait()

            @pl.when(c == 0)
            def _(r0p=r0p):
                pltpu.make_async_copy(
                    ubuf, u_hbm.at[pl.ds(r0p, MROWS)], ssem.at[1]).wait()
        for cp in loads:
            cp.wait()

        def merge_row(r, carry):
            for c2 in range(0, DIM, 32):
                x = wbuf[0] * pbuf[0, r, pl.ds(c2, 32)]
                x = x + wbuf[1] * pbuf[1, r, pl.ds(c2, 32)]
                x = x + wbuf[2] * pbuf[2, r, pl.ds(c2, 32)]
                x = x + wbuf[3] * pbuf[3, r, pl.ds(c2, 32)]
                ubuf[r, pl.ds(c2, 32)] = x
            return carry

        lax.fori_loop(0, MROWS, merge_row, 0)
        pltpu.async_copy(ubuf, ushr.at[pl.ds(r0, MROWS)], ssem.at[0])

        @pl.when(c == 0)
        def _(r0=r0):
            pltpu.async_copy(ubuf, u_hbm.at[pl.ds(r0, MROWS)], ssem.at[1])

    r0p = t * ZROWS + (nm - 1) * MROWS
    pltpu.make_async_copy(
        ubuf, ushr.at[pl.ds(r0p, MROWS)], ssem.at[0]).wait()

    @pl.when(c == 0)
    def _(r0p=r0p):
        pltpu.make_async_copy(
            ubuf, u_hbm.at[pl.ds(r0p, MROWS)], ssem.at[1]).wait()

    cz0.wait()
    cz1.wait()
    plsc.subcore_barrier()

    for ll in range(2):
        acc = acc0 if ll == 0 else acc1
        layer = c * 2 + ll
        for ig in range(CPT // IG):
            base = t * CPT + ig * IG
            ci0 = pltpu.async_copy(src_hbm.at[layer, pl.ds(base, IG)],
                                   sidx, gsem.at[0])
            ci1 = pltpu.async_copy(dst_hbm.at[layer, pl.ds(base, IG)],
                                   didx, gsem.at[1])
            ci0.wait()
            ci1.wait()

            for b in range(NBUF):
                pltpu.async_copy(ushr.at[sidx.at[b]], rows.at[b],
                                 gsem.at[b])

            def group_body(g, carry, acc=acc):
                for b in range(NBUF):
                    j = g * NBUF + b
                    pltpu.make_async_copy(
                        ushr.at[sidx.at[j]], rows.at[b], gsem.at[b]).wait()
                    pltpu.async_copy(rows.at[b], acc.at[didx.at[j]],
                                     ssem.at[b], add=True)
                for b in range(NBUF):
                    j = g * NBUF + b
                    pltpu.make_async_copy(
                        rows.at[b], acc.at[didx.at[j]], ssem.at[b]).wait()

                    @pl.when(j + NBUF < IG)
                    def _(j=j, b=b):
                        pltpu.async_copy(
                            ushr.at[sidx.at[j + NBUF]], rows.at[b],
                            gsem.at[b])
                return carry

            lax.fori_loop(0, IG // NBUF, group_body, 0)

    plsc.subcore_barrier()
    cd0 = pltpu.async_copy(acc0.at[pl.ds(t * ZROWS, ZROWS)],
                           p_hbm.at[c * 2, pl.ds(t * ZROWS, ZROWS)],
                           gsem.at[0])
    cd1 = pltpu.async_copy(acc1.at[pl.ds(t * ZROWS, ZROWS)],
                           p_hbm.at[c * 2 + 1, pl.ds(t * ZROWS, ZROWS)],
                           gsem.at[1])
    cd0.wait()
    cd1.wait()


_hop_m = functools.partial(
    pl.kernel,
    _hop_m_body,
    out_type=(
        jax.ShapeDtypeStruct((N_LAYERS, ACC_ROWS, DIM), jnp.bfloat16),
        jax.ShapeDtypeStruct((ACC_ROWS, DIM), jnp.bfloat16),
    ),
    mesh=plsc.VectorSubcoreMesh(core_axis_name="c", subcore_axis_name="s"),
    compiler_params=pltpu.CompilerParams(use_tc_tiling_on_sc=False),
    scratch_types=[
        pltpu.VMEM((IG, CHUNK), jnp.int32),
        pltpu.VMEM((IG, CHUNK), jnp.int32),
        pltpu.VMEM((NBUF, CHUNK, DIM), jnp.bfloat16),
        pltpu.VMEM((N_LAYERS, MROWS, DIM), jnp.bfloat16),
        pltpu.VMEM((MROWS, DIM), jnp.bfloat16),
        pltpu.VMEM((N_LAYERS, 32), jnp.bfloat16),
        pltpu.VMEM_SHARED((ACC_ROWS, DIM), jnp.bfloat16),
        pltpu.VMEM_SHARED((ACC_ROWS, DIM), jnp.bfloat16),
        pltpu.VMEM_SHARED((ACC_ROWS, DIM), jnp.bfloat16),
        pltpu.SemaphoreType.DMA((NBUF,)),
        pltpu.SemaphoreType.DMA((NBUF,)),
    ],
)()


def _merge_body(p_ref, wb_ref, o_ref):
    p = p_ref[...].astype(jnp.float32)
    o_ref[...] = (wb_ref[0, 0] * p[0] + wb_ref[1, 0] * p[1]
                  + wb_ref[2, 0] * p[2]
                  + wb_ref[3, 0] * p[3]).astype(jnp.bfloat16)


def _merge(P, wb):
    blk = 1024
    return pl.pallas_call(
        _merge_body,
        grid=(ACC_ROWS // blk,),
        in_specs=[
            pl.BlockSpec((N_LAYERS, blk, DIM), lambda i: (0, i, 0)),
            pl.BlockSpec(memory_space=pltpu.SMEM),
        ],
        out_specs=pl.BlockSpec((blk, DIM), lambda i: (i, 0)),
        out_shape=jax.ShapeDtypeStruct((ACC_ROWS, DIM), jnp.bfloat16),
    )(P, wb)


def _merge_final_body(p_ref, wb_ref, wa_ref, u1_ref, u2_ref, o_ref):
    p = p_ref[...].astype(jnp.float32)
    u3 = (wb_ref[0, 0] * p[0] + wb_ref[1, 0] * p[1]
          + wb_ref[2, 0] * p[2] + wb_ref[3, 0] * p[3])
    o_ref[...] = (wa_ref[0, 0] * u1_ref[...].astype(jnp.float32)
                  + wa_ref[1, 0] * u2_ref[...].astype(jnp.float32)
                  + wa_ref[2, 0] * u3)


def _merge_final(P, wb, wa, U1, U2):
    blk = 2000
    return pl.pallas_call(
        _merge_final_body,
        grid=(N // blk,),
        in_specs=[
            pl.BlockSpec((N_LAYERS, blk, DIM), lambda i: (0, i, 0)),
            pl.BlockSpec(memory_space=pltpu.SMEM),
            pl.BlockSpec(memory_space=pltpu.SMEM),
            pl.BlockSpec((blk, DIM), lambda i: (i, 0)),
            pl.BlockSpec((blk, DIM), lambda i: (i, 0)),
        ],
        out_specs=pl.BlockSpec((blk, DIM), lambda i: (i, 0)),
        out_shape=jax.ShapeDtypeStruct((N, DIM), jnp.float32),
    )(P, wb, wa, U1, U2)


def kernel(feature, edge_index, weight_b, weight_a, G):
    src = edge_index[:, 0, :].reshape(N_LAYERS, CHUNKS_PER_LAYER, CHUNK)
    dst = edge_index[:, 1, :].reshape(N_LAYERS, CHUNKS_PER_LAYER, CHUNK)
    pad = CHUNKS_PAD - CHUNKS_PER_LAYER
    src = jnp.pad(src, ((0, 0), (0, pad), (0, 0)))
    dst = jnp.pad(dst, ((0, 0), (0, pad), (0, 0)), constant_values=N)
    zeros = jnp.zeros((ZROWS, DIM), jnp.bfloat16)

    wbv = jnp.broadcast_to(
        weight_b.astype(jnp.bfloat16).reshape(N_LAYERS, 1), (N_LAYERS, 32))

    U = jnp.pad(_project(feature, G), ((0, ACC_ROWS - N), (0, 0)))
    P1 = _hop(U, src, dst, zeros)
    P2, U1 = _hop_m(P1, wbv, src, dst, zeros)
    P3, U2 = _hop_m(P2, wbv, src, dst, zeros)
    return _merge_final(P3, weight_b, weight_a, U1, U2)
